# Initial kernel scaffold; baseline (speedup 1.0000x reference)
import jax
import jax.numpy as jnp
from jax.experimental import pallas as pl


def kernel(embeddings, labels):
    raise NotImplementedError("write your pallas kernel here")


# speedup vs baseline: 5.0438x; 5.0438x over previous
import jax
import jax.numpy as jnp
from jax.experimental import pallas as pl
from jax.experimental.pallas import tpu as pltpu

_ALPHA = 0.2
_BETA = 1.2
_DIST_THR = 0.5
_INF = 1000000.0
_PD_EPS = 1e-4

_N = 1024
_D = 512
_BLOCK_R = 256
_NBLK = _N // _BLOCK_R

_MAX_FINITE_BITS = 0x7F7FFFFF


def _body(emb_blk, emb_full, lab_col_full, lab_col_blk, lab_row, out, acc):
    i = pl.program_id(0)

    @pl.when(i == 0)
    def _init():
        acc[0] = 0.0
        acc[1] = 0.0

    eb = emb_blk[...]
    ef = emb_full[...]
    lr = lab_row[...]
    lcf = lab_col_full[...]
    lcb = lab_col_blk[...]

    eq_full = (lcf == lr).astype(jnp.int32)
    pos_total = jnp.sum(eq_full) - _N
    num_neg = jnp.maximum(1, pos_total // _N)

    prod = jax.lax.dot_general(eb, ef, (((1,), (1,)), ((), ())),
                               preferred_element_type=jnp.float32)
    ones_row = jnp.ones((1, _D), dtype=jnp.float32)
    nr = jax.lax.dot_general(eb * eb, ones_row, (((1,), (1,)), ((), ())),
                             preferred_element_type=jnp.float32)
    nc = jax.lax.dot_general(ones_row, ef * ef, (((1,), (1,)), ((), ())),
                             preferred_element_type=jnp.float32)
    d2 = nr + nc - 2.0 * prod
    d = jnp.sqrt(jnp.clip(d2, _PD_EPS, None))

    gr = i * _BLOCK_R + jax.lax.broadcasted_iota(jnp.int32, (_BLOCK_R, _N), 0)
    gc = jax.lax.broadcasted_iota(jnp.int32, (_BLOCK_R, _N), 1)

    posf = jnp.where((lcb == lr) & (gr != gc), 1.0, 0.0)
    ind = jnp.where(posf > 0.0, 1.0, 0.0) + jnp.where(d < _DIST_THR, 1.0, 0.0)
    masked = d + _INF * ind

    x = jax.lax.bitcast_convert_type(masked, jnp.int32)

    lo = jnp.zeros((_BLOCK_R, 1), jnp.int32)
    hi = jnp.full((_BLOCK_R, 1), _MAX_FINITE_BITS, jnp.int32)

    def _bs_val(_, carry):
        lo_, hi_ = carry
        mid = lo_ + (hi_ - lo_) // 2
        cnt = jnp.sum((x <= mid).astype(jnp.int32), axis=1, keepdims=True)
        ge = cnt >= num_neg
        return jnp.where(ge, lo_, mid + 1), jnp.where(ge, mid, hi_)

    lo, hi = jax.lax.fori_loop(0, 31, _bs_val, (lo, hi))
    t = lo

    cnt_lt = jnp.sum((x < t).astype(jnp.int32), axis=1, keepdims=True)
    m = num_neg - cnt_lt
    tie = x == t

    lo2 = jnp.zeros((_BLOCK_R, 1), jnp.int32)
    hi2 = jnp.full((_BLOCK_R, 1), _N - 1, jnp.int32)

    def _bs_idx(_, carry):
        lo_, hi_ = carry
        mid = lo_ + (hi_ - lo_) // 2
        cnt = jnp.sum((tie & (gc <= mid)).astype(jnp.int32), axis=1,
                      keepdims=True)
        ge = cnt >= m
        return jnp.where(ge, lo_, mid + 1), jnp.where(ge, mid, hi_)

    lo2, hi2 = jax.lax.fori_loop(0, 10, _bs_idx, (lo2, hi2))

    neg = (x < t) | (tie & (gc <= lo2))
    union = (posf > 0.0) | neg
    loss = jnp.maximum(0.0, _ALPHA + (posf * 2.0 - 1.0) * (d - _BETA))
    msel = (union & (loss > 0.0)).astype(jnp.float32)

    acc[0] = acc[0] + jnp.sum(msel * loss)
    acc[1] = acc[1] + jnp.sum(msel)

    @pl.when(i == _NBLK - 1)
    def _fin():
        out[0, 0] = acc[0] / acc[1]


def _run(embeddings, lab_col, lab_row, *, interpret=False):
    grid_spec = pltpu.PrefetchScalarGridSpec(
        num_scalar_prefetch=0,
        grid=(_NBLK,),
        in_specs=[
            pl.BlockSpec((_BLOCK_R, _D), lambda i: (i, 0)),
            pl.BlockSpec((_N, _D), lambda i: (0, 0)),
            pl.BlockSpec((_N, 1), lambda i: (0, 0)),
            pl.BlockSpec((_BLOCK_R, 1), lambda i: (i, 0)),
            pl.BlockSpec((1, _N), lambda i: (0, 0)),
        ],
        out_specs=pl.BlockSpec(memory_space=pltpu.SMEM),
        scratch_shapes=[pltpu.SMEM((2,), jnp.float32)],
    )
    return pl.pallas_call(
        _body,
        grid_spec=grid_spec,
        out_shape=jax.ShapeDtypeStruct((1, 1), jnp.float32),
        interpret=interpret,
    )(embeddings, embeddings, lab_col, lab_col, lab_row)


def kernel(embeddings, labels):
    lab_col = labels.reshape(_N, 1)
    lab_row = labels.reshape(1, _N)
    res = _run(embeddings, lab_col, lab_row)
    return res[0, 0]


# speedup vs baseline: 8.5949x; 1.7041x over previous
import jax
import jax.numpy as jnp
from jax.experimental import pallas as pl
from jax.experimental.pallas import tpu as pltpu

_ALPHA = 0.2
_BETA = 1.2
_DIST_THR = 0.5
_INF = 1000000.0
_PD_EPS = 1e-4

_N = 1024
_D = 512
_BLOCK_R = 256
_NBLK = _N // _BLOCK_R

_MAX_FINITE_BITS = 0x7F7FFFFF


def _body(emb_blk, emb_full, lab_col_full, lab_row_blk, lab_row_full,
          out, acc, kref):
    i = pl.program_id(0)

    @pl.when(i == 0)
    def _init():
        acc[0] = 0.0
        acc[1] = 0.0
        eq_full = (lab_col_full[...] == lab_row_full[...]).astype(jnp.int32)
        pos_total = jnp.sum(eq_full) - _N
        kref[0] = jnp.maximum(1, pos_total // _N)

    num_neg = kref[0]

    eb = emb_blk[...]
    ef = emb_full[...]
    lrb = lab_row_blk[...]
    lcf = lab_col_full[...]

    prod = jax.lax.dot_general(ef, eb, (((1,), (1,)), ((), ())),
                               preferred_element_type=jnp.float32)
    ones_row = jnp.ones((1, _D), dtype=jnp.float32)
    nc = jax.lax.dot_general(ef * ef, ones_row, (((1,), (1,)), ((), ())),
                             preferred_element_type=jnp.float32)
    nr = jax.lax.dot_general(ones_row, eb * eb, (((1,), (1,)), ((), ())),
                             preferred_element_type=jnp.float32)
    d2 = nc + nr - 2.0 * prod
    d = jnp.sqrt(jnp.clip(d2, _PD_EPS, None))

    gj = jax.lax.broadcasted_iota(jnp.int32, (_N, _BLOCK_R), 0)
    gr = i * _BLOCK_R + jax.lax.broadcasted_iota(jnp.int32, (_N, _BLOCK_R), 1)

    posf = jnp.where((lcf == lrb) & (gj != gr), 1.0, 0.0)
    ind = jnp.where(posf > 0.0, 1.0, 0.0) + jnp.where(d < _DIST_THR, 1.0, 0.0)
    masked = d + _INF * ind

    x = jax.lax.bitcast_convert_type(masked, jnp.int32)

    lo = jnp.zeros((1, _BLOCK_R), jnp.int32)
    hi = jnp.full((1, _BLOCK_R), _MAX_FINITE_BITS, jnp.int32)

    def _bs_val(_, carry):
        lo_, hi_ = carry
        mid = lo_ + (hi_ - lo_) // 2
        cnt = jnp.sum((x <= mid).astype(jnp.int32), axis=0, keepdims=True)
        ge = cnt >= num_neg
        return jnp.where(ge, lo_, mid + 1), jnp.where(ge, mid, hi_)

    lo, hi = jax.lax.fori_loop(0, 31, _bs_val, (lo, hi))
    t = lo

    cnt_lt = jnp.sum((x < t).astype(jnp.int32), axis=0, keepdims=True)
    m = num_neg - cnt_lt
    tie = x == t

    lo2 = jnp.zeros((1, _BLOCK_R), jnp.int32)
    hi2 = jnp.full((1, _BLOCK_R), _N - 1, jnp.int32)

    def _bs_idx(_, carry):
        lo_, hi_ = carry
        mid = lo_ + (hi_ - lo_) // 2
        cnt = jnp.sum((tie & (gj <= mid)).astype(jnp.int32), axis=0,
                      keepdims=True)
        ge = cnt >= m
        return jnp.where(ge, lo_, mid + 1), jnp.where(ge, mid, hi_)

    lo2, hi2 = jax.lax.fori_loop(0, 10, _bs_idx, (lo2, hi2))

    neg = (x < t) | (tie & (gj <= lo2))
    union = (posf > 0.0) | neg
    loss = jnp.maximum(0.0, _ALPHA + (posf * 2.0 - 1.0) * (d - _BETA))
    msel = (union & (loss > 0.0)).astype(jnp.float32)

    acc[0] = acc[0] + jnp.sum(msel * loss)
    acc[1] = acc[1] + jnp.sum(msel)

    @pl.when(i == _NBLK - 1)
    def _fin():
        out[0, 0] = acc[0] / acc[1]


def _run(embeddings, lab_col, lab_row, *, interpret=False):
    grid_spec = pltpu.PrefetchScalarGridSpec(
        num_scalar_prefetch=0,
        grid=(_NBLK,),
        in_specs=[
            pl.BlockSpec((_BLOCK_R, _D), lambda i: (i, 0)),
            pl.BlockSpec((_N, _D), lambda i: (0, 0)),
            pl.BlockSpec((_N, 1), lambda i: (0, 0)),
            pl.BlockSpec((1, _BLOCK_R), lambda i: (0, i)),
            pl.BlockSpec((1, _N), lambda i: (0, 0)),
        ],
        out_specs=pl.BlockSpec(memory_space=pltpu.SMEM),
        scratch_shapes=[pltpu.SMEM((2,), jnp.float32),
                        pltpu.SMEM((1,), jnp.int32)],
    )
    return pl.pallas_call(
        _body,
        grid_spec=grid_spec,
        out_shape=jax.ShapeDtypeStruct((1, 1), jnp.float32),
        interpret=interpret,
    )(embeddings, embeddings, lab_col, lab_row, lab_row)


def kernel(embeddings, labels):
    lab_col = labels.reshape(_N, 1)
    lab_row = labels.reshape(1, _N)
    res = _run(embeddings, lab_col, lab_row)
    return res[0, 0]


# speedup vs baseline: 9.9513x; 1.1578x over previous
import jax
import jax.numpy as jnp
from jax.experimental import pallas as pl
from jax.experimental.pallas import tpu as pltpu

_ALPHA = 0.2
_BETA = 1.2
_DIST_THR = 0.5
_INF = 1000000.0
_PD_EPS = 1e-4

_N = 1024
_D = 512
_BLOCK_R = 256
_NBLK = _N // _BLOCK_R

_MAX_FINITE_BITS = 0x7F7FFFFF


def _body(emb_blk, emb_full, lab_col_full, lab_row_blk, lab_row_full,
          out, acc, kref):
    i = pl.program_id(0)

    @pl.when(i == 0)
    def _init():
        acc[0] = 0.0
        acc[1] = 0.0
        eq_full = (lab_col_full[...] == lab_row_full[...]).astype(jnp.int32)
        pos_total = jnp.sum(eq_full) - _N
        kref[0] = jnp.maximum(1, pos_total // _N)

    num_neg = kref[0]

    eb = emb_blk[...]
    ef = emb_full[...]
    lrb = lab_row_blk[...]
    lcf = lab_col_full[...]

    prod = jax.lax.dot_general(ef, eb, (((1,), (1,)), ((), ())),
                               preferred_element_type=jnp.float32)
    ones_row = jnp.ones((1, _D), dtype=jnp.float32)
    nc = jax.lax.dot_general(ef * ef, ones_row, (((1,), (1,)), ((), ())),
                             preferred_element_type=jnp.float32)
    nr = jax.lax.dot_general(ones_row, eb * eb, (((1,), (1,)), ((), ())),
                             preferred_element_type=jnp.float32)
    d2 = nc + nr - 2.0 * prod
    d = jnp.sqrt(jnp.clip(d2, _PD_EPS, None))

    gj = jax.lax.broadcasted_iota(jnp.int32, (_N, _BLOCK_R), 0)
    gr = i * _BLOCK_R + jax.lax.broadcasted_iota(jnp.int32, (_N, _BLOCK_R), 1)

    posf = jnp.where((lcf == lrb) & (gj != gr), 1.0, 0.0)
    ind = jnp.where(posf > 0.0, 1.0, 0.0) + jnp.where(d < _DIST_THR, 1.0, 0.0)
    masked = d + _INF * ind

    x = jax.lax.bitcast_convert_type(masked, jnp.int32)
    xh = (x >> 16).astype(jnp.int16)
    xl = ((x & 0xFFFF) - 32768).astype(jnp.int16)

    def _count16(mask16):
        c = mask16.astype(jnp.int16)
        n = c.shape[0]
        while n > 16:
            n //= 2
            c = c[:n, :] + c[n:, :]
        return jnp.sum(c.astype(jnp.int32), axis=0, keepdims=True)

    lo = jnp.zeros((1, _BLOCK_R), jnp.int32)
    hi = jnp.full((1, _BLOCK_R), 32767, jnp.int32)

    def _bs_hi(_, carry):
        lo_, hi_ = carry
        mid = lo_ + (hi_ - lo_) // 2
        ge = _count16(xh <= mid.astype(jnp.int16)) >= num_neg
        return jnp.where(ge, lo_, mid + 1), jnp.where(ge, mid, hi_)

    lo, hi = jax.lax.fori_loop(0, 15, _bs_hi, (lo, hi))
    th = lo.astype(jnp.int16)

    hlt = xh < th
    eqm = xh == th
    need = num_neg - _count16(hlt)

    lo2 = jnp.full((1, _BLOCK_R), -32768, jnp.int32)
    hi2 = jnp.full((1, _BLOCK_R), 32767, jnp.int32)

    def _bs_lo(_, carry):
        lo_, hi_ = carry
        mid = lo_ + (hi_ - lo_) // 2
        ge = _count16(eqm & (xl <= mid.astype(jnp.int16))) >= need
        return jnp.where(ge, lo_, mid + 1), jnp.where(ge, mid, hi_)

    lo2, hi2 = jax.lax.fori_loop(0, 16, _bs_lo, (lo2, hi2))
    tl = lo2.astype(jnp.int16)

    llt = eqm & (xl < tl)
    tiem = eqm & (xl == tl)
    m = need - _count16(llt)

    gj16 = gj.astype(jnp.int16)
    lo3 = jnp.zeros((1, _BLOCK_R), jnp.int32)
    hi3 = jnp.full((1, _BLOCK_R), _N - 1, jnp.int32)

    def _bs_idx(_, carry):
        lo_, hi_ = carry
        mid = lo_ + (hi_ - lo_) // 2
        ge = _count16(tiem & (gj16 <= mid.astype(jnp.int16))) >= m
        return jnp.where(ge, lo_, mid + 1), jnp.where(ge, mid, hi_)

    lo3, hi3 = jax.lax.fori_loop(0, 10, _bs_idx, (lo3, hi3))

    neg = hlt | llt | (tiem & (gj16 <= lo3.astype(jnp.int16)))
    union = (posf > 0.0) | neg
    loss = jnp.maximum(0.0, _ALPHA + (posf * 2.0 - 1.0) * (d - _BETA))
    msel = (union & (loss > 0.0)).astype(jnp.float32)

    acc[0] = acc[0] + jnp.sum(msel * loss)
    acc[1] = acc[1] + jnp.sum(msel)

    @pl.when(i == _NBLK - 1)
    def _fin():
        out[0, 0] = acc[0] / acc[1]


def _run(embeddings, lab_col, lab_row, *, interpret=False):
    grid_spec = pltpu.PrefetchScalarGridSpec(
        num_scalar_prefetch=0,
        grid=(_NBLK,),
        in_specs=[
            pl.BlockSpec((_BLOCK_R, _D), lambda i: (i, 0)),
            pl.BlockSpec((_N, _D), lambda i: (0, 0)),
            pl.BlockSpec((_N, 1), lambda i: (0, 0)),
            pl.BlockSpec((1, _BLOCK_R), lambda i: (0, i)),
            pl.BlockSpec((1, _N), lambda i: (0, 0)),
        ],
        out_specs=pl.BlockSpec(memory_space=pltpu.SMEM),
        scratch_shapes=[pltpu.SMEM((2,), jnp.float32),
                        pltpu.SMEM((1,), jnp.int32)],
    )
    return pl.pallas_call(
        _body,
        grid_spec=grid_spec,
        out_shape=jax.ShapeDtypeStruct((1, 1), jnp.float32),
        interpret=interpret,
    )(embeddings, embeddings, lab_col, lab_row, lab_row)


def kernel(embeddings, labels):
    lab_col = labels.reshape(_N, 1)
    lab_row = labels.reshape(1, _N)
    res = _run(embeddings, lab_col, lab_row)
    return res[0, 0]


# speedup vs baseline: 10.4023x; 1.0453x over previous
import jax
import jax.numpy as jnp
from jax.experimental import pallas as pl
from jax.experimental.pallas import tpu as pltpu

_ALPHA = 0.2
_BETA = 1.2
_DIST_THR = 0.5
_INF = 1000000.0
_PD_EPS = 1e-4

_N = 1024
_D = 512
_BLOCK_R = 512
_NBLK = _N // _BLOCK_R

_MAX_FINITE_BITS = 0x7F7FFFFF


def _body(emb_blk, emb_full, lab_col_full, lab_row_blk, lab_row_full,
          out, acc, kref):
    i = pl.program_id(0)

    @pl.when(i == 0)
    def _init():
        acc[0] = 0.0
        acc[1] = 0.0
        eq_full = (lab_col_full[...] == lab_row_full[...]).astype(jnp.int32)
        pos_total = jnp.sum(eq_full) - _N
        kref[0] = jnp.maximum(1, pos_total // _N)

    num_neg = kref[0]

    eb = emb_blk[...]
    ef = emb_full[...]
    lrb = lab_row_blk[...]
    lcf = lab_col_full[...]

    prod = jax.lax.dot_general(ef, eb, (((1,), (1,)), ((), ())),
                               preferred_element_type=jnp.float32)
    ones_row = jnp.ones((1, _D), dtype=jnp.float32)
    nc = jax.lax.dot_general(ef * ef, ones_row, (((1,), (1,)), ((), ())),
                             preferred_element_type=jnp.float32)
    nr = jax.lax.dot_general(ones_row, eb * eb, (((1,), (1,)), ((), ())),
                             preferred_element_type=jnp.float32)
    d2 = nc + nr - 2.0 * prod
    d = jnp.sqrt(jnp.clip(d2, _PD_EPS, None))

    gj = jax.lax.broadcasted_iota(jnp.int32, (_N, _BLOCK_R), 0)
    gr = i * _BLOCK_R + jax.lax.broadcasted_iota(jnp.int32, (_N, _BLOCK_R), 1)

    posf = jnp.where((lcf == lrb) & (gj != gr), 1.0, 0.0)
    ind = jnp.where(posf > 0.0, 1.0, 0.0) + jnp.where(d < _DIST_THR, 1.0, 0.0)
    masked = d + _INF * ind

    x = jax.lax.bitcast_convert_type(masked, jnp.int32)
    xh = (x >> 16).astype(jnp.int16)
    xl = ((x & 0xFFFF) - 32768).astype(jnp.int16)

    def _count16(mask16):
        c = mask16.astype(jnp.int16)
        n = c.shape[0]
        while n > 16:
            n //= 2
            c = c[:n, :] + c[n:, :]
        return jnp.sum(c.astype(jnp.int32), axis=0, keepdims=True)

    lo = jnp.zeros((1, _BLOCK_R), jnp.int32)
    hi = jnp.full((1, _BLOCK_R), 32767, jnp.int32)

    def _bs_hi(_, carry):
        lo_, hi_ = carry
        mid = lo_ + (hi_ - lo_) // 2
        ge = _count16(xh <= mid.astype(jnp.int16)) >= num_neg
        return jnp.where(ge, lo_, mid + 1), jnp.where(ge, mid, hi_)

    lo, hi = jax.lax.fori_loop(0, 15, _bs_hi, (lo, hi))
    th = lo.astype(jnp.int16)

    hlt = xh < th
    eqm = xh == th
    need = num_neg - _count16(hlt)

    lo2 = jnp.full((1, _BLOCK_R), -32768, jnp.int32)
    hi2 = jnp.full((1, _BLOCK_R), 32767, jnp.int32)

    def _bs_lo(_, carry):
        lo_, hi_ = carry
        mid = lo_ + (hi_ - lo_) // 2
        ge = _count16(eqm & (xl <= mid.astype(jnp.int16))) >= need
        return jnp.where(ge, lo_, mid + 1), jnp.where(ge, mid, hi_)

    lo2, hi2 = jax.lax.fori_loop(0, 16, _bs_lo, (lo2, hi2))
    tl = lo2.astype(jnp.int16)

    llt = eqm & (xl < tl)
    tiem = eqm & (xl == tl)
    m = need - _count16(llt)

    gj16 = gj.astype(jnp.int16)
    lo3 = jnp.zeros((1, _BLOCK_R), jnp.int32)
    hi3 = jnp.full((1, _BLOCK_R), _N - 1, jnp.int32)

    def _bs_idx(_, carry):
        lo_, hi_ = carry
        mid = lo_ + (hi_ - lo_) // 2
        ge = _count16(tiem & (gj16 <= mid.astype(jnp.int16))) >= m
        return jnp.where(ge, lo_, mid + 1), jnp.where(ge, mid, hi_)

    lo3, hi3 = jax.lax.fori_loop(0, 10, _bs_idx, (lo3, hi3))

    neg = hlt | llt | (tiem & (gj16 <= lo3.astype(jnp.int16)))
    union = (posf > 0.0) | neg
    loss = jnp.maximum(0.0, _ALPHA + (posf * 2.0 - 1.0) * (d - _BETA))
    msel = (union & (loss > 0.0)).astype(jnp.float32)

    acc[0] = acc[0] + jnp.sum(msel * loss)
    acc[1] = acc[1] + jnp.sum(msel)

    @pl.when(i == _NBLK - 1)
    def _fin():
        out[0, 0] = acc[0] / acc[1]


def _run(embeddings, lab_col, lab_row, *, interpret=False):
    grid_spec = pltpu.PrefetchScalarGridSpec(
        num_scalar_prefetch=0,
        grid=(_NBLK,),
        in_specs=[
            pl.BlockSpec((_BLOCK_R, _D), lambda i: (i, 0)),
            pl.BlockSpec((_N, _D), lambda i: (0, 0)),
            pl.BlockSpec((_N, 1), lambda i: (0, 0)),
            pl.BlockSpec((1, _BLOCK_R), lambda i: (0, i)),
            pl.BlockSpec((1, _N), lambda i: (0, 0)),
        ],
        out_specs=pl.BlockSpec(memory_space=pltpu.SMEM),
        scratch_shapes=[pltpu.SMEM((2,), jnp.float32),
                        pltpu.SMEM((1,), jnp.int32)],
    )
    return pl.pallas_call(
        _body,
        grid_spec=grid_spec,
        out_shape=jax.ShapeDtypeStruct((1, 1), jnp.float32),
        interpret=interpret,
    )(embeddings, embeddings, lab_col, lab_row, lab_row)


def kernel(embeddings, labels):
    lab_col = labels.reshape(_N, 1)
    lab_row = labels.reshape(1, _N)
    res = _run(embeddings, lab_col, lab_row)
    return res[0, 0]
